# baseline (speedup 1.0000x reference)
import functools

import jax
import jax.numpy as jnp
from jax import lax
from jax.experimental import pallas as pl
from jax.experimental.pallas import tpu as pltpu
from jax.experimental.pallas import tpu_sc as plsc

N = 10000
E = 320000
D = 128
H = 256
O = 128

NPAD = 10240
DUMMY = NPAD
SROWS = 10368
ZR = SROWS // 16
WR = NPAD // 16
C = 64
NB = 4
EPAD = 327680
G2 = EPAD // (16 * C)
BN = 512

HD = 5120
HROWS = 5128
C1 = 32
G1 = EPAD // (16 * C1)
TS = 624


def _agg1_body(x_hbm, src_hbm, dst2_hbm, z128_hbm, zc_hbm, ones_hbm,
               sum_out, cnt_out, *refs):
    srcs = refs[0:2]
    dsts = refs[2:4]
    rows = refs[4:6]
    ones_v = refs[6]
    bounce = refs[7]
    tab_sh = refs[8]
    acc_sh = refs[9]
    cnt_sh = refs[10]
    isems = refs[11:13]
    gsems = refs[13:15]
    c = lax.axis_index("c")
    s = lax.axis_index("s")
    sbs = s * (G1 * C1)
    ebd = c * EPAD + s * (G1 * C1)

    @pl.when(s < 15)
    def _t0():
        pltpu.sync_copy(x_hbm.at[pl.ds(s * TS, TS)],
                        tab_sh.at[pl.ds(s * TS, TS)])

    @pl.when(s == 15)
    def _t1():
        pltpu.sync_copy(x_hbm.at[pl.ds(15 * TS, N - 15 * TS)],
                        tab_sh.at[pl.ds(15 * TS, N - 15 * TS)])

    @pl.when(s < 15)
    def _z0():
        pltpu.sync_copy(z128_hbm.at[pl.ds(0, 320)],
                        acc_sh.at[pl.ds(s * 320, 320)])

    @pl.when(s == 15)
    def _z1():
        pltpu.sync_copy(z128_hbm, acc_sh.at[pl.ds(15 * 320, 328)])

    pltpu.sync_copy(zc_hbm, bounce)

    @pl.when(s < 15)
    def _z2():
        pltpu.sync_copy(bounce.at[pl.ds(0, 320)],
                        cnt_sh.at[pl.ds(s * 320, 320)])

    @pl.when(s == 15)
    def _z3():
        pltpu.sync_copy(bounce, cnt_sh.at[pl.ds(15 * 320, 328)])

    pltpu.sync_copy(ones_hbm, ones_v)
    plsc.subcore_barrier()

    def idx_dma(g, b):
        pltpu.async_copy(src_hbm.at[pl.ds(sbs + g * C1, C1)], srcs[b],
                         isems[b])
        pltpu.async_copy(dst2_hbm.at[pl.ds(ebd + g * C1, C1)], dsts[b],
                         isems[b])

    def idx_wait(g, b):
        pltpu.make_async_copy(
            src_hbm.at[pl.ds(sbs + g * C1, C1)], srcs[b], isems[b]).wait()
        pltpu.make_async_copy(
            dst2_hbm.at[pl.ds(ebd + g * C1, C1)], dsts[b], isems[b]).wait()

    idx_dma(0, 0)
    idx_dma(1, 1)
    idx_wait(0, 0)
    pltpu.async_copy(tab_sh.at[srcs[0]], rows[0], gsems[0])

    def body(go, carry):
        for b in range(2):
            g = go * 2 + b
            bp = 1 - b

            @pl.when(g + 1 < G1)
            def _fire():
                idx_wait(g + 1, bp)
                pltpu.async_copy(tab_sh.at[srcs[bp]], rows[bp], gsems[bp])

            pltpu.make_async_copy(tab_sh.at[srcs[b]], rows[b],
                                  gsems[b]).wait()
            pltpu.sync_copy(rows[b], acc_sh.at[dsts[b]], add=True)
            pltpu.sync_copy(ones_v, cnt_sh.at[dsts[b]], add=True)

            @pl.when(g + 2 < G1)
            def _pre():
                idx_dma(g + 2, b)

        return carry

    lax.fori_loop(0, G1 // 2, body, 0)
    plsc.subcore_barrier()
    pltpu.sync_copy(acc_sh.at[pl.ds(s * 320, 320)],
                    sum_out.at[pl.ds(c * HD + s * 320, 320)])
    pltpu.sync_copy(cnt_sh.at[pl.ds(s * 320, 320)], bounce.at[pl.ds(0, 320)])
    pltpu.sync_copy(bounce.at[pl.ds(0, 320)],
                    cnt_out.at[pl.ds(c * HD + s * 320, 320)])


_agg1 = functools.partial(
    pl.kernel,
    mesh=plsc.VectorSubcoreMesh(core_axis_name="c", subcore_axis_name="s"),
    out_type=[
        jax.ShapeDtypeStruct((NPAD, 128), jnp.float32),
        jax.ShapeDtypeStruct((NPAD,), jnp.float32),
    ],
    scratch_types=(
        [pltpu.VMEM((C1,), jnp.int32)] * 4
        + [pltpu.VMEM((C1, 128), jnp.float32)] * 2
        + [pltpu.VMEM((C1,), jnp.float32),
           pltpu.VMEM((328,), jnp.float32),
           pltpu.VMEM_SHARED((N, 128), jnp.float32),
           pltpu.VMEM_SHARED((HROWS, 128), jnp.float32),
           pltpu.VMEM_SHARED((HROWS,), jnp.float32)]
        + [pltpu.SemaphoreType.DMA] * 4
    ),
)(_agg1_body)


def _agg2_body(htab_hbm, src2_hbm, dst_hbm, z128_hbm,
               sum_out, *refs):
    srcs = refs[0:NB]
    dsts = refs[NB:2 * NB]
    rows = refs[2 * NB:3 * NB]
    sum_sh = refs[3 * NB]
    isems = refs[3 * NB + 1:3 * NB + 1 + NB]
    gsems = refs[3 * NB + 1 + NB:3 * NB + 1 + 2 * NB]
    c = lax.axis_index("c")
    s = lax.axis_index("s")
    sb = c * EPAD + s * (G2 * C)
    db = s * (G2 * C)
    pltpu.sync_copy(z128_hbm, sum_sh.at[pl.ds(s * ZR, ZR)])
    plsc.subcore_barrier()

    def idx_dma(g, b):
        pltpu.async_copy(src2_hbm.at[pl.ds(sb + g * C, C)], srcs[b], isems[b])
        pltpu.async_copy(dst_hbm.at[pl.ds(db + g * C, C)], dsts[b], isems[b])

    def idx_wait(g, b):
        pltpu.make_async_copy(
            src2_hbm.at[pl.ds(sb + g * C, C)], srcs[b], isems[b]).wait()
        pltpu.make_async_copy(
            dst_hbm.at[pl.ds(db + g * C, C)], dsts[b], isems[b]).wait()

    for b in range(NB):
        idx_dma(b, b)
    for b in range(NB - 1):
        idx_wait(b, b)
        pltpu.async_copy(htab_hbm.at[srcs[b]], rows[b], gsems[b])

    def body(go, carry):
        for b in range(NB):
            g = go * NB + b
            bp = (b + NB - 1) % NB

            @pl.when(g + NB - 1 < G2)
            def _fire():
                idx_wait(g + NB - 1, bp)
                pltpu.async_copy(htab_hbm.at[srcs[bp]], rows[bp], gsems[bp])

            pltpu.make_async_copy(htab_hbm.at[srcs[b]], rows[b],
                                  gsems[b]).wait()
            pltpu.sync_copy(rows[b], sum_sh.at[dsts[b]], add=True)

            @pl.when(g + NB < G2)
            def _pre():
                idx_dma(g + NB, b)

        return carry

    lax.fori_loop(0, G2 // NB, body, 0)
    plsc.subcore_barrier()
    pltpu.sync_copy(sum_sh.at[pl.ds(s * WR, WR)],
                    sum_out.at[pl.ds(c * NPAD + s * WR, WR)])


_agg2 = functools.partial(
    pl.kernel,
    mesh=plsc.VectorSubcoreMesh(core_axis_name="c", subcore_axis_name="s"),
    out_type=jax.ShapeDtypeStruct((2 * NPAD, 128), jnp.float32),
    scratch_types=(
        [pltpu.VMEM((C,), jnp.int32)] * (2 * NB)
        + [pltpu.VMEM((C, 128), jnp.float32)] * NB
        + [pltpu.VMEM_SHARED((SROWS, 128), jnp.float32)]
        + [pltpu.SemaphoreType.DMA] * (2 * NB)
    ),
)(_agg2_body)


def _tc1_body(s_ref, c_ref, x_ref, wl_ref, wr_ref, b_ref, o_ref):
    rc = 1.0 / jnp.maximum(c_ref[...], 1.0)
    aggr = s_ref[...] * rc
    z = (jnp.dot(aggr, wl_ref[...], preferred_element_type=jnp.float32)
         + jnp.dot(x_ref[...], wr_ref[...], preferred_element_type=jnp.float32)
         + b_ref[...])
    h = jnp.maximum(z, 0.0)
    o_ref[0] = h[:, :128]
    o_ref[1] = h[:, 128:]


def _tc2_body(s_ref, c_ref, h_ref, w2l_ref, w2r_ref, b2_ref,
              wlin_ref, blin_ref, o_ref):
    rc = 1.0 / jnp.maximum(c_ref[...], 1.0)
    z = (jnp.dot(s_ref[0] * rc, w2l_ref[:128], preferred_element_type=jnp.float32)
         + jnp.dot(s_ref[1] * rc, w2l_ref[128:], preferred_element_type=jnp.float32)
         + jnp.dot(h_ref[0], w2r_ref[:128], preferred_element_type=jnp.float32)
         + jnp.dot(h_ref[1], w2r_ref[128:], preferred_element_type=jnp.float32)
         + b2_ref[...])
    hh = jnp.maximum(z, 0.0)
    o_ref[...] = (jnp.dot(hh, wlin_ref[...], preferred_element_type=jnp.float32)
                  + blin_ref[...])


def _tc1(sum1, cnt, x_pad, W1l, W1r, b1):
    return pl.pallas_call(
        _tc1_body,
        grid=(NPAD // BN,),
        in_specs=[
            pl.BlockSpec((BN, 128), lambda i: (i, 0)),
            pl.BlockSpec((BN, 1), lambda i: (i, 0)),
            pl.BlockSpec((BN, 128), lambda i: (i, 0)),
            pl.BlockSpec((128, 256), lambda i: (0, 0)),
            pl.BlockSpec((128, 256), lambda i: (0, 0)),
            pl.BlockSpec((1, 256), lambda i: (0, 0)),
        ],
        out_specs=pl.BlockSpec((2, BN, 128), lambda i: (0, i, 0)),
        out_shape=jax.ShapeDtypeStruct((2, NPAD, 128), jnp.float32),
    )(sum1, cnt, x_pad, W1l, W1r, b1)


def _tc2(sum2, cnt, htab, W2l, W2r, b2, Wlin, blin):
    return pl.pallas_call(
        _tc2_body,
        grid=(NPAD // BN,),
        in_specs=[
            pl.BlockSpec((2, BN, 128), lambda i: (0, i, 0)),
            pl.BlockSpec((BN, 1), lambda i: (i, 0)),
            pl.BlockSpec((2, BN, 128), lambda i: (0, i, 0)),
            pl.BlockSpec((256, 256), lambda i: (0, 0)),
            pl.BlockSpec((256, 256), lambda i: (0, 0)),
            pl.BlockSpec((1, 256), lambda i: (0, 0)),
            pl.BlockSpec((256, 128), lambda i: (0, 0)),
            pl.BlockSpec((1, 128), lambda i: (0, 0)),
        ],
        out_specs=pl.BlockSpec((BN, 128), lambda i: (i, 0)),
        out_shape=jax.ShapeDtypeStruct((NPAD, 128), jnp.float32),
    )(sum2, cnt, htab, W2l, W2r, b2, Wlin, blin)


def kernel(x, edge_index, W1l, b1, W1r, W2l, b2, W2r, Wlin, blin):
    src = edge_index[0].astype(jnp.int32)
    dst = edge_index[1].astype(jnp.int32)
    srcp = jnp.concatenate([src, jnp.zeros((EPAD - E,), jnp.int32)])
    dstp = jnp.concatenate([dst, jnp.full((EPAD - E,), DUMMY, jnp.int32)])
    src2 = jnp.concatenate([srcp, srcp + NPAD])
    dst2 = jnp.concatenate([jnp.where(dstp < HD, dstp, HD),
                            jnp.where(dstp >= HD, dstp - HD, HD)])
    x_pad = jnp.pad(x, ((0, NPAD - N), (0, 0)))
    z128 = jnp.zeros((ZR, 128), jnp.float32)
    z328 = jnp.zeros((328, 128), jnp.float32)
    zc = jnp.zeros((328,), jnp.float32)
    ones = jnp.ones((C1,), jnp.float32)

    sum1, cnt = _agg1(x, srcp, dst2, z328, zc, ones)
    cnt = cnt.reshape(NPAD, 1)

    htab = _tc1(sum1, cnt, x_pad, W1l, W1r, b1.reshape(1, H))

    sum2 = _agg2(htab.reshape(2 * NPAD, 128), src2, dstp, z128)
    sum2 = sum2.reshape(2, NPAD, 128)

    out = _tc2(sum2, cnt, htab, W2l, W2r, b2.reshape(1, H),
               Wlin, blin.reshape(1, O))
    return out[:N]


# speedup vs baseline: 1.1183x; 1.1183x over previous
import functools

import jax
import jax.numpy as jnp
from jax import lax
from jax.experimental import pallas as pl
from jax.experimental.pallas import tpu as pltpu
from jax.experimental.pallas import tpu_sc as plsc

N = 10000
E = 320000
D = 128
H = 256
O = 128

NPAD = 10240
DUMMY = NPAD
SROWS = 10368
ZR = SROWS // 16
WR = NPAD // 16
CROWS = 10496
CW = 8
CR8 = CROWS * CW // 16
C = 64
NB = 4
EPAD = 327680
G1 = EPAD // (32 * C)
G2 = EPAD // (16 * C)
BN = 512


def _agg1_body(x_hbm, src_hbm, dst_hbm, d8_hbm, z128_hbm, zc_hbm, ones_hbm,
               sum_out, cnt_out, *refs):
    srcs = refs[0:NB]
    dsts = refs[NB:2 * NB]
    d8s = refs[2 * NB:3 * NB]
    rows = refs[3 * NB:4 * NB]
    ones_v = refs[4 * NB]
    bounce = refs[4 * NB + 1]
    sum_sh = refs[4 * NB + 2]
    cnt_sh = refs[4 * NB + 3]
    isems = refs[4 * NB + 4:4 * NB + 4 + NB]
    gsems = refs[4 * NB + 4 + NB:4 * NB + 4 + 2 * NB]
    c = lax.axis_index("c")
    s = lax.axis_index("s")
    wid = s * 2 + c
    eb = wid * (G1 * C)
    pltpu.sync_copy(z128_hbm, sum_sh.at[pl.ds(s * ZR, ZR)])
    pltpu.sync_copy(zc_hbm, bounce)
    pltpu.sync_copy(bounce, cnt_sh.at[pl.ds(s * CR8, CR8)])
    pltpu.sync_copy(ones_hbm, ones_v)
    plsc.subcore_barrier()

    def idx_dma(g, b):
        pltpu.async_copy(src_hbm.at[pl.ds(eb + g * C, C)], srcs[b], isems[b])
        pltpu.async_copy(dst_hbm.at[pl.ds(eb + g * C, C)], dsts[b], isems[b])
        pltpu.async_copy(d8_hbm.at[pl.ds(eb + g * C, C)], d8s[b], isems[b])

    def idx_wait(g, b):
        pltpu.make_async_copy(
            src_hbm.at[pl.ds(eb + g * C, C)], srcs[b], isems[b]).wait()
        pltpu.make_async_copy(
            dst_hbm.at[pl.ds(eb + g * C, C)], dsts[b], isems[b]).wait()
        pltpu.make_async_copy(
            d8_hbm.at[pl.ds(eb + g * C, C)], d8s[b], isems[b]).wait()

    for b in range(NB):
        idx_dma(b, b)
    for b in range(NB - 1):
        idx_wait(b, b)
        pltpu.async_copy(x_hbm.at[srcs[b]], rows[b], gsems[b])

    def body(go, carry):
        for b in range(NB):
            g = go * NB + b
            bp = (b + NB - 1) % NB

            @pl.when(g + NB - 1 < G1)
            def _fire():
                idx_wait(g + NB - 1, bp)
                pltpu.async_copy(x_hbm.at[srcs[bp]], rows[bp], gsems[bp])

            pltpu.make_async_copy(x_hbm.at[srcs[b]], rows[b], gsems[b]).wait()
            pltpu.sync_copy(rows[b], sum_sh.at[dsts[b]], add=True)
            pltpu.sync_copy(ones_v, cnt_sh.at[d8s[b]], add=True)

            @pl.when(g + NB < G1)
            def _pre():
                idx_dma(g + NB, b)

        return carry

    lax.fori_loop(0, G1 // NB, body, 0)
    plsc.subcore_barrier()
    pltpu.sync_copy(sum_sh.at[pl.ds(s * WR, WR)],
                    sum_out.at[pl.ds(c * NPAD + s * WR, WR)])
    pltpu.sync_copy(cnt_sh.at[pl.ds(s * CR8, CR8)], bounce)
    pltpu.sync_copy(bounce, cnt_out.at[pl.ds(c * CROWS * CW + s * CR8, CR8)])


_agg1 = functools.partial(
    pl.kernel,
    mesh=plsc.VectorSubcoreMesh(core_axis_name="c", subcore_axis_name="s"),
    out_type=[
        jax.ShapeDtypeStruct((2 * NPAD, 128), jnp.float32),
        jax.ShapeDtypeStruct((2 * CROWS * CW,), jnp.float32),
    ],
    scratch_types=(
        [pltpu.VMEM((C,), jnp.int32)] * (3 * NB)
        + [pltpu.VMEM((C, 128), jnp.float32)] * NB
        + [pltpu.VMEM((C,), jnp.float32),
           pltpu.VMEM((CR8,), jnp.float32),
           pltpu.VMEM_SHARED((SROWS, 128), jnp.float32),
           pltpu.VMEM_SHARED((CROWS * CW,), jnp.float32)]
        + [pltpu.SemaphoreType.DMA] * (2 * NB)
    ),
)(_agg1_body)


def _agg2_body(htab_hbm, src2_hbm, dst_hbm, z128_hbm,
               sum_out, *refs):
    srcs = refs[0:NB]
    dsts = refs[NB:2 * NB]
    rows = refs[2 * NB:3 * NB]
    sum_sh = refs[3 * NB]
    isems = refs[3 * NB + 1:3 * NB + 1 + NB]
    gsems = refs[3 * NB + 1 + NB:3 * NB + 1 + 2 * NB]
    c = lax.axis_index("c")
    s = lax.axis_index("s")
    sb = c * EPAD + s * (G2 * C)
    db = s * (G2 * C)
    pltpu.sync_copy(z128_hbm, sum_sh.at[pl.ds(s * ZR, ZR)])
    plsc.subcore_barrier()

    def idx_dma(g, b):
        pltpu.async_copy(src2_hbm.at[pl.ds(sb + g * C, C)], srcs[b], isems[b])
        pltpu.async_copy(dst_hbm.at[pl.ds(db + g * C, C)], dsts[b], isems[b])

    def idx_wait(g, b):
        pltpu.make_async_copy(
            src2_hbm.at[pl.ds(sb + g * C, C)], srcs[b], isems[b]).wait()
        pltpu.make_async_copy(
            dst_hbm.at[pl.ds(db + g * C, C)], dsts[b], isems[b]).wait()

    for b in range(NB):
        idx_dma(b, b)
    for b in range(NB - 1):
        idx_wait(b, b)
        pltpu.async_copy(htab_hbm.at[srcs[b]], rows[b], gsems[b])

    def body(go, carry):
        for b in range(NB):
            g = go * NB + b
            bp = (b + NB - 1) % NB

            @pl.when(g + NB - 1 < G2)
            def _fire():
                idx_wait(g + NB - 1, bp)
                pltpu.async_copy(htab_hbm.at[srcs[bp]], rows[bp], gsems[bp])

            pltpu.make_async_copy(htab_hbm.at[srcs[b]], rows[b],
                                  gsems[b]).wait()
            pltpu.sync_copy(rows[b], sum_sh.at[dsts[b]], add=True)

            @pl.when(g + NB < G2)
            def _pre():
                idx_dma(g + NB, b)

        return carry

    lax.fori_loop(0, G2 // NB, body, 0)
    plsc.subcore_barrier()
    pltpu.sync_copy(sum_sh.at[pl.ds(s * WR, WR)],
                    sum_out.at[pl.ds(c * NPAD + s * WR, WR)])


_agg2 = functools.partial(
    pl.kernel,
    mesh=plsc.VectorSubcoreMesh(core_axis_name="c", subcore_axis_name="s"),
    out_type=jax.ShapeDtypeStruct((2 * NPAD, 128), jnp.float32),
    scratch_types=(
        [pltpu.VMEM((C,), jnp.int32)] * (2 * NB)
        + [pltpu.VMEM((C, 128), jnp.float32)] * NB
        + [pltpu.VMEM_SHARED((SROWS, 128), jnp.float32)]
        + [pltpu.SemaphoreType.DMA] * (2 * NB)
    ),
)(_agg2_body)


def _tc1_body(s_ref, c_ref, x_ref, wl_ref, wr_ref, b_ref, o_ref):
    cnt = c_ref[0][:, :1] + c_ref[1][:, :1]
    rc = 1.0 / jnp.maximum(cnt, 1.0)
    aggr = (s_ref[0] + s_ref[1]) * rc
    z = (jnp.dot(aggr, wl_ref[...], preferred_element_type=jnp.float32)
         + jnp.dot(x_ref[...], wr_ref[...], preferred_element_type=jnp.float32)
         + b_ref[...])
    h = jnp.maximum(z, 0.0)
    o_ref[0] = h[:, :128]
    o_ref[1] = h[:, 128:]


def _tc2_body(s_ref, c_ref, h_ref, w2l_ref, w2r_ref, b2_ref,
              wlin_ref, blin_ref, o_ref):
    cnt = c_ref[0][:, :1] + c_ref[1][:, :1]
    rc = 1.0 / jnp.maximum(cnt, 1.0)
    z = (jnp.dot(s_ref[0] * rc, w2l_ref[:128], preferred_element_type=jnp.float32)
         + jnp.dot(s_ref[1] * rc, w2l_ref[128:], preferred_element_type=jnp.float32)
         + jnp.dot(h_ref[0], w2r_ref[:128], preferred_element_type=jnp.float32)
         + jnp.dot(h_ref[1], w2r_ref[128:], preferred_element_type=jnp.float32)
         + b2_ref[...])
    hh = jnp.maximum(z, 0.0)
    o_ref[...] = (jnp.dot(hh, wlin_ref[...], preferred_element_type=jnp.float32)
                  + blin_ref[...])


def _tc1(sum1, cnt, x_pad, W1l, W1r, b1):
    return pl.pallas_call(
        _tc1_body,
        grid=(NPAD // BN,),
        in_specs=[
            pl.BlockSpec((2, BN, 128), lambda i: (0, i, 0)),
            pl.BlockSpec((2, BN, CW), lambda i: (0, i, 0)),
            pl.BlockSpec((BN, 128), lambda i: (i, 0)),
            pl.BlockSpec((128, 256), lambda i: (0, 0)),
            pl.BlockSpec((128, 256), lambda i: (0, 0)),
            pl.BlockSpec((1, 256), lambda i: (0, 0)),
        ],
        out_specs=pl.BlockSpec((2, BN, 128), lambda i: (0, i, 0)),
        out_shape=jax.ShapeDtypeStruct((2, NPAD, 128), jnp.float32),
    )(sum1, cnt, x_pad, W1l, W1r, b1)


def _tc2(sum2, cnt, htab, W2l, W2r, b2, Wlin, blin):
    return pl.pallas_call(
        _tc2_body,
        grid=(NPAD // BN,),
        in_specs=[
            pl.BlockSpec((2, BN, 128), lambda i: (0, i, 0)),
            pl.BlockSpec((2, BN, CW), lambda i: (0, i, 0)),
            pl.BlockSpec((2, BN, 128), lambda i: (0, i, 0)),
            pl.BlockSpec((256, 256), lambda i: (0, 0)),
            pl.BlockSpec((256, 256), lambda i: (0, 0)),
            pl.BlockSpec((1, 256), lambda i: (0, 0)),
            pl.BlockSpec((256, 128), lambda i: (0, 0)),
            pl.BlockSpec((1, 128), lambda i: (0, 0)),
        ],
        out_specs=pl.BlockSpec((BN, 128), lambda i: (i, 0)),
        out_shape=jax.ShapeDtypeStruct((NPAD, 128), jnp.float32),
    )(sum2, cnt, htab, W2l, W2r, b2, Wlin, blin)


def kernel(x, edge_index, W1l, b1, W1r, W2l, b2, W2r, Wlin, blin):
    src = edge_index[0].astype(jnp.int32)
    dst = edge_index[1].astype(jnp.int32)
    srcp = jnp.concatenate([src, jnp.zeros((EPAD - E,), jnp.int32)])
    dstp = jnp.concatenate([dst, jnp.full((EPAD - E,), DUMMY, jnp.int32)])
    src2 = jnp.concatenate([srcp, srcp + NPAD])
    d8p = dstp * CW
    x_pad = jnp.pad(x, ((0, NPAD - N), (0, 0)))
    z128 = jnp.zeros((ZR, 128), jnp.float32)
    zc = jnp.zeros((CR8,), jnp.float32)
    ones = jnp.ones((C,), jnp.float32)

    sum1, cnt = _agg1(x_pad, srcp, dstp, d8p, z128, zc, ones)
    sum1 = sum1.reshape(2, NPAD, 128)
    cnt = cnt.reshape(2, CROWS, CW)

    htab = _tc1(sum1, cnt, x_pad, W1l, W1r, b1.reshape(1, H))

    sum2 = _agg2(htab.reshape(2 * NPAD, 128), src2, dstp, z128)
    sum2 = sum2.reshape(2, NPAD, 128)

    out = _tc2(sum2, cnt, htab, W2l, W2r, b2.reshape(1, H),
               Wlin, blin.reshape(1, O))
    return out[:N]
